# baseline (speedup 1.0000x reference)
import jax
import jax.numpy as jnp
from jax import lax
from jax.experimental import pallas as pl
from jax.experimental.pallas import tpu as pltpu
from jax.experimental.pallas import tpu_sc as plsc
import functools

RES_F = 100000.0
VOCAB = 100000
EMBED = 64
NW = 32
ROWS_PER_W = 512
N_CHUNK = 8
CHUNK = 128

TILE = 128
N_TILES = VOCAB // TILE
TAIL = VOCAB - N_TILES * TILE
MAX_TPW = (N_TILES + NW - 1) // NW

_MESH = plsc.VectorSubcoreMesh(core_axis_name="c", subcore_axis_name="s")


@functools.partial(
    pl.kernel,
    mesh=_MESH,
    out_type=(
        jax.ShapeDtypeStruct((NW * N_CHUNK, CHUNK), jnp.int32),
        jax.ShapeDtypeStruct((VOCAB * EMBED,), jnp.float32),
    ),
    scratch_types=[
        pltpu.VMEM((2, ROWS_PER_W), jnp.float32),
        pltpu.VMEM((N_CHUNK, CHUNK), jnp.int32),
        pltpu.VMEM((EMBED, TILE + 1), jnp.float32),
        pltpu.VMEM((TILE * EMBED,), jnp.float32),
        pltpu.VMEM((TAIL * EMBED,), jnp.float32),
    ],
    compiler_params=pltpu.CompilerParams(needs_layout_passes=False),
)
def _sc_prepare(xt_hbm, tablet_hbm, tail_hbm, idx_hbm, lin_hbm,
                xv, idxv, tr, trt, tailv):
    wid = lax.axis_index("s") * 2 + lax.axis_index("c")
    nb = wid * ROWS_PER_W

    pltpu.sync_copy(xt_hbm.at[:, pl.ds(nb, ROWS_PER_W)], xv)
    for j in range(N_CHUNK // 2):
        def body(i, _):
            off = j * CHUNK + i * 16
            idxv[j, pl.ds(i * 16, 16)] = (
                xv[0, pl.ds(off, 16)] * RES_F).astype(jnp.int32)
            idxv[j + N_CHUNK // 2, pl.ds(i * 16, 16)] = (
                xv[1, pl.ds(off, 16)] * RES_F).astype(jnp.int32)
            return 0
        lax.fori_loop(0, CHUNK // 16, body, 0)
    pltpu.sync_copy(idxv, idx_hbm.at[pl.ds(wid * N_CHUNK, N_CHUNK)])

    lanes = lax.iota(jnp.int32, 16)
    cols16 = [lanes + cc * 16 for cc in range(EMBED // 16)]
    UNROLL = 8

    def do_tile(t, _):
        tile_id = wid + t * NW

        @pl.when(tile_id < N_TILES)
        def _():
            rb = tile_id * TILE
            pltpu.sync_copy(tablet_hbm.at[:, pl.ds(rb, TILE)], tr.at[:, pl.ds(0, TILE)])

            def rows(rr, _):
                for u in range(UNROLL):
                    r = rr * UNROLL + u
                    rv = jnp.full((16,), 0, jnp.int32) + r
                    base = r * EMBED
                    for cc in range(EMBED // 16):
                        v = plsc.load_gather(tr, [cols16[cc], rv])
                        trt[pl.ds(base + cc * 16, 16)] = v
                return 0
            lax.fori_loop(0, TILE // UNROLL, rows, 0)
            pltpu.sync_copy(trt, lin_hbm.at[pl.ds(rb * EMBED, TILE * EMBED)])
        return 0

    lax.fori_loop(0, MAX_TPW, do_tile, 0)

    @pl.when(wid == NW - 1)
    def _():
        pltpu.sync_copy(tail_hbm, tailv)
        pltpu.sync_copy(tailv,
                        lin_hbm.at[pl.ds(N_TILES * TILE * EMBED, TAIL * EMBED)])


@functools.partial(
    pl.kernel,
    mesh=_MESH,
    out_type=jax.ShapeDtypeStruct((NW * ROWS_PER_W, 2 * EMBED), jnp.float32),
    scratch_types=[
        pltpu.VMEM((N_CHUNK, CHUNK), jnp.int32),
        pltpu.VMEM((2 * ROWS_PER_W, EMBED), jnp.float32),
        pltpu.SemaphoreType.DMA,
    ],
    compiler_params=pltpu.CompilerParams(
        use_tc_tiling_on_sc=False, needs_layout_passes=False),
)
def _sc_gather(idx_hbm, table_hbm, out_hbm, idxv, rows, sem):
    wid = lax.axis_index("s") * 2 + lax.axis_index("c")
    nb = wid * ROWS_PER_W

    pltpu.sync_copy(idx_hbm.at[pl.ds(wid * N_CHUNK, N_CHUNK)], idxv)

    copies = []
    for j in range(N_CHUNK):
        copies.append(
            pltpu.async_copy(
                table_hbm.at[idxv.at[j]],
                rows.at[pl.ds(j * CHUNK, CHUNK)],
                sem,
            )
        )
    for c in copies:
        c.wait()

    pltpu.sync_copy(rows.at[pl.ds(0, ROWS_PER_W)],
                    out_hbm.at[pl.ds(nb, ROWS_PER_W), pl.ds(0, EMBED)])
    pltpu.sync_copy(rows.at[pl.ds(ROWS_PER_W, ROWS_PER_W)],
                    out_hbm.at[pl.ds(nb, ROWS_PER_W), pl.ds(EMBED, EMBED)])


def kernel(x, table):
    tail = table[N_TILES * TILE:].reshape(-1)
    idx, lin = _sc_prepare(x.T, table.T, tail)
    return _sc_gather(idx, lin.reshape(VOCAB, EMBED))


# speedup vs baseline: 2.4340x; 2.4340x over previous
import jax
import jax.numpy as jnp
from jax import lax
from jax.experimental import pallas as pl
from jax.experimental.pallas import tpu as pltpu
from jax.experimental.pallas import tpu_sc as plsc
import functools

RES_F = 100000.0
VOCAB = 100000
EMBED = 64
NW = 32
ROWS_PER_W = 512
N_CHUNK = 8
CHUNK = 128

TL = 1664
TGRID = (VOCAB + TL - 1) // TL

_MESH = plsc.VectorSubcoreMesh(core_axis_name="c", subcore_axis_name="s")


def _tc_relayout_body(tt_ref, out_ref):
    t = tt_ref[...].T
    out_ref[:, 0:EMBED] = t
    out_ref[:, EMBED:2 * EMBED] = t


_tc_relayout = pl.pallas_call(
    _tc_relayout_body,
    out_shape=jax.ShapeDtypeStruct((VOCAB, 2 * EMBED), jnp.float32),
    grid=(TGRID,),
    in_specs=[pl.BlockSpec((EMBED, TL), lambda g: (0, g))],
    out_specs=pl.BlockSpec((TL, 2 * EMBED), lambda g: (g, 0)),
)


@functools.partial(
    pl.kernel,
    mesh=_MESH,
    out_type=jax.ShapeDtypeStruct((NW * ROWS_PER_W, 2 * EMBED), jnp.float32),
    scratch_types=[
        pltpu.VMEM((2, ROWS_PER_W), jnp.float32),
        pltpu.VMEM((N_CHUNK, CHUNK), jnp.int32),
        pltpu.VMEM((ROWS_PER_W, 2 * EMBED), jnp.float32),
        pltpu.SemaphoreType.DMA,
    ],
    compiler_params=pltpu.CompilerParams(
        use_tc_tiling_on_sc=False, needs_layout_passes=False),
)
def _sc_embed(xt_hbm, table_hbm, out_hbm, xv, idxv, rows, sem):
    wid = lax.axis_index("s") * 2 + lax.axis_index("c")
    nb = wid * ROWS_PER_W

    pltpu.sync_copy(xt_hbm.at[:, pl.ds(nb, ROWS_PER_W)], xv)
    for j in range(N_CHUNK // 2):
        def body(i, _):
            off = j * CHUNK + i * 16
            idxv[j, pl.ds(i * 16, 16)] = (
                xv[0, pl.ds(off, 16)] * RES_F).astype(jnp.int32)
            idxv[j + N_CHUNK // 2, pl.ds(i * 16, 16)] = (
                xv[1, pl.ds(off, 16)] * RES_F).astype(jnp.int32)
            return 0
        lax.fori_loop(0, CHUNK // 16, body, 0)

    for h in range(2):
        copies = []
        for j in range(N_CHUNK // 2):
            copies.append(
                pltpu.async_copy(
                    table_hbm.at[idxv.at[h * (N_CHUNK // 2) + j]],
                    rows.at[pl.ds(j * CHUNK, CHUNK)],
                    sem,
                )
            )
        for c in copies:
            c.wait()
        pltpu.sync_copy(
            rows.at[:, pl.ds(h * EMBED, EMBED)],
            out_hbm.at[pl.ds(nb, ROWS_PER_W), pl.ds(h * EMBED, EMBED)])


def kernel(x, table):
    dup = _tc_relayout(table.T)
    return _sc_embed(x.T, dup)


# speedup vs baseline: 3.1830x; 1.3077x over previous
import jax
import jax.numpy as jnp
from jax import lax
from jax.experimental import pallas as pl
from jax.experimental.pallas import tpu as pltpu
from jax.experimental.pallas import tpu_sc as plsc
import functools

RES_F = 100000.0
VOCAB = 100000
EMBED = 64
NW = 32
ROWS_PER_W = 512
N_CHUNK = 8
CHUNK = 128

TL = 3328
TGRID = (VOCAB + TL - 1) // TL
VPAD = TGRID * TL

_MESH = plsc.VectorSubcoreMesh(core_axis_name="c", subcore_axis_name="s")


def _tc_relayout_body(tt_ref, out_ref):
    t = tt_ref[...].T
    out_ref[...] = jnp.concatenate([t, t], axis=1)


_tc_relayout = pl.pallas_call(
    _tc_relayout_body,
    out_shape=jax.ShapeDtypeStruct((VPAD, 2 * EMBED), jnp.float32),
    grid=(TGRID,),
    in_specs=[pl.BlockSpec((EMBED, TL), lambda g: (0, g))],
    out_specs=pl.BlockSpec((TL, 2 * EMBED), lambda g: (g, 0)),
)


@functools.partial(
    pl.kernel,
    mesh=_MESH,
    out_type=jax.ShapeDtypeStruct((NW * ROWS_PER_W, 2 * EMBED), jnp.float32),
    scratch_types=[
        pltpu.VMEM((2, ROWS_PER_W), jnp.float32),
        pltpu.VMEM((N_CHUNK, CHUNK), jnp.int32),
        pltpu.VMEM((2 * ROWS_PER_W, EMBED), jnp.float32),
        pltpu.SemaphoreType.DMA,
    ],
    compiler_params=pltpu.CompilerParams(
        use_tc_tiling_on_sc=False, needs_layout_passes=False),
)
def _sc_embed(xt_hbm, table_hbm, out_hbm, xv, idxv, rows, sem):
    wid = lax.axis_index("s") * 2 + lax.axis_index("c")
    nb = wid * ROWS_PER_W

    pltpu.sync_copy(xt_hbm.at[:, pl.ds(nb, ROWS_PER_W)], xv)
    for j in range(N_CHUNK // 2):
        def body(i, _):
            off = j * CHUNK + i * 16
            idxv[j, pl.ds(i * 16, 16)] = (
                xv[0, pl.ds(off, 16)] * RES_F).astype(jnp.int32) * 2
            idxv[j + N_CHUNK // 2, pl.ds(i * 16, 16)] = (
                xv[1, pl.ds(off, 16)] * RES_F).astype(jnp.int32) * 2
            return 0
        lax.fori_loop(0, CHUNK // 16, body, 0)

    copies = []
    for j in range(N_CHUNK):
        copies.append(
            pltpu.async_copy(
                table_hbm.at[idxv.at[j]],
                rows.at[pl.ds(j * CHUNK, CHUNK)],
                sem,
            )
        )
    for c in copies:
        c.wait()

    pltpu.sync_copy(rows.at[pl.ds(0, ROWS_PER_W)],
                    out_hbm.at[pl.ds(nb, ROWS_PER_W), pl.ds(0, EMBED)])
    pltpu.sync_copy(rows.at[pl.ds(ROWS_PER_W, ROWS_PER_W)],
                    out_hbm.at[pl.ds(nb, ROWS_PER_W), pl.ds(EMBED, EMBED)])


def kernel(x, table):
    dup = _tc_relayout(table.T)
    return _sc_embed(x.T, dup.reshape(2 * VPAD, EMBED))


# speedup vs baseline: 3.1840x; 1.0003x over previous
import jax
import jax.numpy as jnp
from jax import lax
from jax.experimental import pallas as pl
from jax.experimental.pallas import tpu as pltpu
from jax.experimental.pallas import tpu_sc as plsc
import functools

RES_F = 100000.0
VOCAB = 100000
EMBED = 64
NW = 32
ROWS_PER_W = 512
N_CHUNK = 8
CHUNK = 128

TL = 3328
TGRID = (VOCAB + TL - 1) // TL
VPAD = TGRID * TL

_MESH = plsc.VectorSubcoreMesh(core_axis_name="c", subcore_axis_name="s")


def _tc_relayout_body(tt_ref, out_ref):
    t = tt_ref[...].T
    out_ref[...] = jnp.concatenate([t, t], axis=1)


_tc_relayout = pl.pallas_call(
    _tc_relayout_body,
    out_shape=jax.ShapeDtypeStruct((VPAD, 2 * EMBED), jnp.float32),
    grid=(TGRID,),
    in_specs=[pl.BlockSpec((EMBED, TL), lambda g: (0, g))],
    out_specs=pl.BlockSpec((TL, 2 * EMBED), lambda g: (g, 0)),
    compiler_params=pltpu.CompilerParams(
        dimension_semantics=("parallel",)),
)


@functools.partial(
    pl.kernel,
    mesh=_MESH,
    out_type=jax.ShapeDtypeStruct((NW * ROWS_PER_W, 2 * EMBED), jnp.float32),
    scratch_types=[
        pltpu.VMEM((2, ROWS_PER_W), jnp.float32),
        pltpu.VMEM((N_CHUNK, CHUNK), jnp.int32),
        pltpu.VMEM((2 * ROWS_PER_W, EMBED), jnp.float32),
        pltpu.SemaphoreType.DMA,
    ],
    compiler_params=pltpu.CompilerParams(
        use_tc_tiling_on_sc=False, needs_layout_passes=False),
)
def _sc_embed(xt_hbm, table_hbm, out_hbm, xv, idxv, rows, sem):
    wid = lax.axis_index("s") * 2 + lax.axis_index("c")
    nb = wid * ROWS_PER_W

    pltpu.sync_copy(xt_hbm.at[:, pl.ds(nb, ROWS_PER_W)], xv)
    for j in range(N_CHUNK // 2):
        def body(i, _):
            off = j * CHUNK + i * 16
            idxv[j, pl.ds(i * 16, 16)] = (
                xv[0, pl.ds(off, 16)] * RES_F).astype(jnp.int32) * 2
            idxv[j + N_CHUNK // 2, pl.ds(i * 16, 16)] = (
                xv[1, pl.ds(off, 16)] * RES_F).astype(jnp.int32) * 2
            return 0
        lax.fori_loop(0, CHUNK // 16, body, 0)

    copies = []
    for j in range(N_CHUNK):
        copies.append(
            pltpu.async_copy(
                table_hbm.at[idxv.at[j]],
                rows.at[pl.ds(j * CHUNK, CHUNK)],
                sem,
            )
        )
    for c in copies:
        c.wait()

    pltpu.sync_copy(rows.at[pl.ds(0, ROWS_PER_W)],
                    out_hbm.at[pl.ds(nb, ROWS_PER_W), pl.ds(0, EMBED)])
    pltpu.sync_copy(rows.at[pl.ds(ROWS_PER_W, ROWS_PER_W)],
                    out_hbm.at[pl.ds(nb, ROWS_PER_W), pl.ds(EMBED, EMBED)])


def kernel(x, table):
    dup = _tc_relayout(table.T)
    return _sc_embed(x.T, dup.reshape(2 * VPAD, EMBED))
